# baseline (speedup 1.0000x reference)
import functools

import jax
import jax.numpy as jnp
from jax.experimental import pallas as pl
from jax.experimental.pallas import tpu as pltpu

BATCH = 4096
NSTEP = 11
LENGTH = 20
INPUT_SIZE = 64

BLK = 128
NBLK = BATCH // BLK


def _tc_gather(weight, steps):
    def body(s_ref, weight_hbm, out_vmem, sem):
        i = pl.program_id(0)
        base = i * BLK
        copies = []
        for j in range(BLK):
            copies.append(
                pltpu.make_async_copy(
                    weight_hbm.at[base + j, s_ref[base + j]],
                    out_vmem.at[j], sem))
        for c in copies:
            c.start()
        for c in copies:
            c.wait()

    grid_spec = pltpu.PrefetchScalarGridSpec(
        num_scalar_prefetch=1,
        grid=(NBLK,),
        in_specs=[pl.BlockSpec(memory_space=pl.ANY)],
        out_specs=pl.BlockSpec((BLK, LENGTH, INPUT_SIZE),
                               lambda i, s_ref: (i, 0, 0)),
        scratch_shapes=[pltpu.SemaphoreType.DMA],
    )
    return pl.pallas_call(
        body,
        grid_spec=grid_spec,
        out_shape=jax.ShapeDtypeStruct((BATCH, LENGTH, INPUT_SIZE),
                                       jnp.float32),
    )(steps, weight)


def kernel(embeddings, conditions, steps, weight, lengths):
    out = _tc_gather(weight, steps.astype(jnp.int32))
    return (out, lengths)
